# baseline (speedup 1.0000x reference)
import functools

import jax
import jax.numpy as jnp
from jax.experimental import pallas as pl
from jax.experimental.pallas import tpu as pltpu


def _fused_kernel(x_ref, bm_ref, w1_ref, b1_ref, w2_ref, b2_ref,
                  wp1_ref, bp1_ref, wp2_ref, bp2_ref, o_ref,
                  w1b, wp1b, w2b, wp2b, *, E):
    @pl.when(pl.program_id(0) == 0)
    def _cast_weights():
        w1b[...] = w1_ref[...].astype(jnp.bfloat16)
        wp1b[...] = wp1_ref[...].astype(jnp.bfloat16)
        w2b[...] = w2_ref[...].astype(jnp.bfloat16)
        wp2b[...] = wp2_ref[...].astype(jnp.bfloat16)

    x = x_ref[...].astype(jnp.bfloat16)
    bm = bm_ref[...]
    h = jnp.dot(x, w1b[...], preferred_element_type=jnp.float32)
    h = jnp.maximum(h + b1_ref[...], 0.0).astype(jnp.bfloat16)
    o = jnp.dot(h, w2b[...], preferred_element_type=jnp.float32)
    for e in range(E):
        he = jnp.dot(x, wp1b[e], preferred_element_type=jnp.float32)
        he = (jnp.maximum(he + bp1_ref[e], 0.0)
              * bm[:, e][:, None]).astype(jnp.bfloat16)
        o = o + jnp.dot(he, wp2b[e], preferred_element_type=jnp.float32)
    o = o + b2_ref[...] + jnp.dot(bm, bp2_ref[...],
                                  preferred_element_type=jnp.float32)
    o_ref[...] = o


def kernel(x, in_bitmap, W1, b1, W2, b2, Wp1, bp1, Wp2, bp2):
    N, D = x.shape
    H = W1.shape[1]
    E, _, PH = Wp1.shape
    C = W2.shape[1]

    bm = in_bitmap.astype(jnp.float32)

    BN = 512
    grid = (N // BN,)
    out = pl.pallas_call(
        functools.partial(_fused_kernel, E=E),
        grid=grid,
        in_specs=[
            pl.BlockSpec((BN, D), lambda i: (i, 0)),
            pl.BlockSpec((BN, E), lambda i: (i, 0)),
            pl.BlockSpec((D, H), lambda i: (0, 0)),
            pl.BlockSpec((1, H), lambda i: (0, 0)),
            pl.BlockSpec((H, C), lambda i: (0, 0)),
            pl.BlockSpec((1, C), lambda i: (0, 0)),
            pl.BlockSpec((E, D, PH), lambda i: (0, 0, 0)),
            pl.BlockSpec((E, PH), lambda i: (0, 0)),
            pl.BlockSpec((E, PH, C), lambda i: (0, 0, 0)),
            pl.BlockSpec((E, C), lambda i: (0, 0)),
        ],
        out_specs=pl.BlockSpec((BN, C), lambda i: (i, 0)),
        out_shape=jax.ShapeDtypeStruct((N, C), jnp.float32),
        scratch_shapes=[
            pltpu.VMEM((D, H), jnp.bfloat16),
            pltpu.VMEM((E, D, PH), jnp.bfloat16),
            pltpu.VMEM((H, C), jnp.bfloat16),
            pltpu.VMEM((E, PH, C), jnp.bfloat16),
        ],
        compiler_params=pltpu.CompilerParams(
            dimension_semantics=("arbitrary",),
        ),
    )(x, bm, W1, b1.reshape(1, H), W2, b2.reshape(1, C), Wp1, bp1, Wp2, bp2)
    return out


# speedup vs baseline: 1.0182x; 1.0182x over previous
import functools

import jax
import jax.numpy as jnp
from jax.experimental import pallas as pl
from jax.experimental.pallas import tpu as pltpu


def _fused_kernel(x_ref, bm_ref, w1_ref, b1_ref, w2_ref, b2_ref,
                  wp1_ref, bp1_ref, wp2_ref, bp2_ref, o_ref, *, E):
    x = x_ref[...]
    bm = bm_ref[...]
    h = jnp.dot(x, w1_ref[...], preferred_element_type=jnp.float32)
    h = jnp.maximum(h + b1_ref[...], 0.0)
    o = jnp.dot(h, w2_ref[...], preferred_element_type=jnp.float32)
    for e in range(E):
        he = jnp.dot(x, wp1_ref[e], preferred_element_type=jnp.float32)
        he = jnp.maximum(he + bp1_ref[e], 0.0) * bm[:, e][:, None]
        o = o + jnp.dot(he, wp2_ref[e], preferred_element_type=jnp.float32)
    o = o + b2_ref[...] + jnp.dot(bm, bp2_ref[...],
                                  preferred_element_type=jnp.float32)
    o_ref[...] = o


def kernel(x, in_bitmap, W1, b1, W2, b2, Wp1, bp1, Wp2, bp2):
    N, D = x.shape
    H = W1.shape[1]
    E, _, PH = Wp1.shape
    C = W2.shape[1]

    bm = in_bitmap.astype(jnp.float32)

    BN = 512
    grid = (N // BN,)
    out = pl.pallas_call(
        functools.partial(_fused_kernel, E=E),
        grid=grid,
        in_specs=[
            pl.BlockSpec((BN, D), lambda i: (i, 0)),
            pl.BlockSpec((BN, E), lambda i: (i, 0)),
            pl.BlockSpec((D, H), lambda i: (0, 0)),
            pl.BlockSpec((1, H), lambda i: (0, 0)),
            pl.BlockSpec((H, C), lambda i: (0, 0)),
            pl.BlockSpec((1, C), lambda i: (0, 0)),
            pl.BlockSpec((E, D, PH), lambda i: (0, 0, 0)),
            pl.BlockSpec((E, PH), lambda i: (0, 0)),
            pl.BlockSpec((E, PH, C), lambda i: (0, 0, 0)),
            pl.BlockSpec((E, C), lambda i: (0, 0)),
        ],
        out_specs=pl.BlockSpec((BN, C), lambda i: (i, 0)),
        out_shape=jax.ShapeDtypeStruct((N, C), jnp.float32),
        compiler_params=pltpu.CompilerParams(
            dimension_semantics=("parallel",),
        ),
    )(x, bm, W1, b1.reshape(1, H), W2, b2.reshape(1, C), Wp1, bp1, Wp2, bp2)
    return out


# speedup vs baseline: 1.0665x; 1.0474x over previous
import functools

import jax
import jax.numpy as jnp
from jax.experimental import pallas as pl
from jax.experimental.pallas import tpu as pltpu


def _fused_kernel(x_ref, bm_ref, w1_ref, b1_ref, w2_ref, b2_ref,
                  wp1_ref, bp1_ref, wp2_ref, bp2_ref, o_ref, *, E):
    x = x_ref[...]
    bm = bm_ref[...]
    h = jnp.dot(x, w1_ref[...], preferred_element_type=jnp.float32)
    h = jnp.maximum(h + b1_ref[...], 0.0)
    o = jnp.dot(h, w2_ref[...], preferred_element_type=jnp.float32)
    for e in range(E):
        he = jnp.dot(x, wp1_ref[e], preferred_element_type=jnp.float32)
        he = jnp.maximum(he + bp1_ref[e], 0.0) * bm[:, e][:, None]
        o = o + jnp.dot(he, wp2_ref[e], preferred_element_type=jnp.float32)
    o = o + b2_ref[...] + jnp.dot(bm, bp2_ref[...],
                                  preferred_element_type=jnp.float32)
    o_ref[...] = o


def kernel(x, in_bitmap, W1, b1, W2, b2, Wp1, bp1, Wp2, bp2):
    N, D = x.shape
    H = W1.shape[1]
    E, _, PH = Wp1.shape
    C = W2.shape[1]

    bm = in_bitmap.astype(jnp.float32)

    BN = 1024
    grid = (N // BN,)
    out = pl.pallas_call(
        functools.partial(_fused_kernel, E=E),
        grid=grid,
        in_specs=[
            pl.BlockSpec((BN, D), lambda i: (i, 0)),
            pl.BlockSpec((BN, E), lambda i: (i, 0)),
            pl.BlockSpec((D, H), lambda i: (0, 0)),
            pl.BlockSpec((1, H), lambda i: (0, 0)),
            pl.BlockSpec((H, C), lambda i: (0, 0)),
            pl.BlockSpec((1, C), lambda i: (0, 0)),
            pl.BlockSpec((E, D, PH), lambda i: (0, 0, 0)),
            pl.BlockSpec((E, PH), lambda i: (0, 0)),
            pl.BlockSpec((E, PH, C), lambda i: (0, 0, 0)),
            pl.BlockSpec((E, C), lambda i: (0, 0)),
        ],
        out_specs=pl.BlockSpec((BN, C), lambda i: (i, 0)),
        out_shape=jax.ShapeDtypeStruct((N, C), jnp.float32),
        compiler_params=pltpu.CompilerParams(
            dimension_semantics=("parallel",),
        ),
    )(x, bm, W1, b1.reshape(1, H), W2, b2.reshape(1, C), Wp1, bp1, Wp2, bp2)
    return out


# speedup vs baseline: 1.0818x; 1.0143x over previous
import functools

import jax
import jax.numpy as jnp
from jax.experimental import pallas as pl
from jax.experimental.pallas import tpu as pltpu


def _fused_kernel(x_ref, bm_ref, w1_ref, b1_ref, w2_ref, b2_ref,
                  wp1_ref, bp1_ref, wp2_ref, bp2_ref, o_ref, *, E):
    x = x_ref[...]
    bm = bm_ref[...]
    h = jnp.dot(x, w1_ref[...], preferred_element_type=jnp.float32)
    h = jnp.maximum(h + b1_ref[...], 0.0)
    o = jnp.dot(h, w2_ref[...], preferred_element_type=jnp.float32)
    for e in range(E):
        he = jnp.dot(x, wp1_ref[e], preferred_element_type=jnp.float32)
        he = jnp.maximum(he + bp1_ref[e], 0.0) * bm[:, e][:, None]
        o = o + jnp.dot(he, wp2_ref[e], preferred_element_type=jnp.float32)
    o = o + b2_ref[...] + jnp.dot(bm, bp2_ref[...],
                                  preferred_element_type=jnp.float32)
    o_ref[...] = o


def kernel(x, in_bitmap, W1, b1, W2, b2, Wp1, bp1, Wp2, bp2):
    N, D = x.shape
    H = W1.shape[1]
    E, _, PH = Wp1.shape
    C = W2.shape[1]

    bm = in_bitmap.astype(jnp.float32)

    BN = 2048
    grid = (N // BN,)
    out = pl.pallas_call(
        functools.partial(_fused_kernel, E=E),
        grid=grid,
        in_specs=[
            pl.BlockSpec((BN, D), lambda i: (i, 0)),
            pl.BlockSpec((BN, E), lambda i: (i, 0)),
            pl.BlockSpec((D, H), lambda i: (0, 0)),
            pl.BlockSpec((1, H), lambda i: (0, 0)),
            pl.BlockSpec((H, C), lambda i: (0, 0)),
            pl.BlockSpec((1, C), lambda i: (0, 0)),
            pl.BlockSpec((E, D, PH), lambda i: (0, 0, 0)),
            pl.BlockSpec((E, PH), lambda i: (0, 0)),
            pl.BlockSpec((E, PH, C), lambda i: (0, 0, 0)),
            pl.BlockSpec((E, C), lambda i: (0, 0)),
        ],
        out_specs=pl.BlockSpec((BN, C), lambda i: (i, 0)),
        out_shape=jax.ShapeDtypeStruct((N, C), jnp.float32),
        compiler_params=pltpu.CompilerParams(
            dimension_semantics=("parallel",),
        ),
    )(x, bm, W1, b1.reshape(1, H), W2, b2.reshape(1, C), Wp1, bp1, Wp2, bp2)
    return out
